# baseline (speedup 1.0000x reference)
import functools

import jax
import jax.numpy as jnp
from jax import lax
from jax.experimental import pallas as pl
from jax.experimental.pallas import tpu as pltpu
from jax.experimental.pallas import tpu_sc as plsc

BATCH = 16384
DIM = 32
NFEAT = 26
NW = 32
BPW = BATCH // NW
IDX_W = 128
ROWS_PER_W = BPW // IDX_W
NPARAM = 80


def _sc_body(uix_hbm, mix_hbm, feat_hbm, utab_hbm, mtab_hbm, par_hbm,
             out_hbm, uix_v, mix_v, u_v, m_v, f_v, p_v, o_v, sem):
    nc = 2
    wid = lax.axis_index("s") * nc + lax.axis_index("c")
    base = wid * BPW
    row0 = wid * ROWS_PER_W

    pltpu.sync_copy(uix_hbm.at[pl.ds(row0, ROWS_PER_W)], uix_v)
    pltpu.sync_copy(mix_hbm.at[pl.ds(row0, ROWS_PER_W)], mix_v)
    pltpu.sync_copy(par_hbm, p_v)

    copies = [pltpu.async_copy(feat_hbm.at[pl.ds(base, BPW)], f_v, sem)]
    for j in range(ROWS_PER_W):
        copies.append(pltpu.async_copy(
            utab_hbm.at[uix_v.at[j]], u_v.at[pl.ds(j * IDX_W, IDX_W)], sem))
        copies.append(pltpu.async_copy(
            mtab_hbm.at[mix_v.at[j]], m_v.at[pl.ds(j * IDX_W, IDX_W)], sem))
    for c in copies:
        c.wait()

    w0 = p_v[pl.ds(0, 16)]
    w1 = p_v[pl.ds(16, 16)]
    wf0 = p_v[pl.ds(32, 16)]
    wf1 = p_v[pl.ds(48, 16)]
    b_vec = p_v[pl.ds(64, 16)]
    lane = lax.iota(jnp.int32, 16)

    def lanesum(t):
        for k in (1, 2, 4, 8):
            t = t + t.at[lane ^ k].get(mode="promise_in_bounds",
                                       unique_indices=True)
        return t

    def group(g, carry):
        base_i = g * 16
        acc = jnp.zeros((16,), jnp.float32)
        for j in range(16):
            i = base_i + j
            u0 = u_v[i, pl.ds(0, 16)]
            u1 = u_v[i, pl.ds(16, 16)]
            m0 = m_v[i, pl.ds(0, 16)]
            m1 = m_v[i, pl.ds(16, 16)]
            f0 = f_v[i, pl.ds(0, 16)]
            f1 = f_v[i, pl.ds(16, 16)]
            t = u0 * m0 * w0 + u1 * m1 * w1 + f0 * wf0 + f1 * wf1
            acc = jnp.where(lane == j, lanesum(t), acc)
        o_v[pl.ds(base_i, 16)] = acc + b_vec
        return carry

    lax.fori_loop(0, BPW // 16, group, 0)

    pltpu.sync_copy(o_v, out_hbm.at[pl.ds(base, BPW)])


_sc_call = functools.partial(
    pl.kernel,
    mesh=plsc.VectorSubcoreMesh(core_axis_name="c", subcore_axis_name="s"),
    out_type=jax.ShapeDtypeStruct((BATCH,), jnp.float32),
    compiler_params=pltpu.CompilerParams(use_tc_tiling_on_sc=False),
    scratch_types=[
        pltpu.VMEM((ROWS_PER_W, IDX_W), jnp.int32),
        pltpu.VMEM((ROWS_PER_W, IDX_W), jnp.int32),
        pltpu.VMEM((BPW, DIM), jnp.float32),
        pltpu.VMEM((BPW, DIM), jnp.float32),
        pltpu.VMEM((BPW, DIM), jnp.float32),
        pltpu.VMEM((NPARAM,), jnp.float32),
        pltpu.VMEM((BPW,), jnp.float32),
        pltpu.SemaphoreType.DMA,
    ],
)(_sc_body)


def kernel(user_idx, movie_idx, features, user_table, movie_table, fc_w, fc_b):
    uix = user_idx.astype(jnp.int32).reshape(BATCH // IDX_W, IDX_W)
    mix = movie_idx.astype(jnp.int32).reshape(BATCH // IDX_W, IDX_W)
    featp = jnp.pad(features, ((0, 0), (0, DIM - NFEAT)))
    w = fc_w.reshape(-1)
    params = jnp.concatenate([
        w[:DIM],
        jnp.pad(w[DIM:], (0, DIM - NFEAT)),
        jnp.broadcast_to(fc_b.reshape(-1), (16,)),
    ])
    return _sc_call(uix, mix, featp, user_table, movie_table, params)


# speedup vs baseline: 1.0018x; 1.0018x over previous
import functools

import jax
import jax.numpy as jnp
from jax import lax
from jax.experimental import pallas as pl
from jax.experimental.pallas import tpu as pltpu
from jax.experimental.pallas import tpu_sc as plsc

BATCH = 16384
DIM = 32
NFEAT = 26
NW = 32
BPW = BATCH // NW
IDX_W = 128
ROWS_PER_W = BPW // IDX_W
NPARAM = 80


def _sc_body(uix_hbm, mix_hbm, feat_hbm, utab_hbm, mtab_hbm, par_hbm,
             out_hbm, uix_v, mix_v, u_v, m_v, f_v, p_v, o_v, sem):
    nc = 2
    wid = lax.axis_index("s") * nc + lax.axis_index("c")
    base = wid * BPW

    pltpu.sync_copy(uix_hbm.at[pl.ds(base, BPW)], uix_v)
    pltpu.sync_copy(mix_hbm.at[pl.ds(base, BPW)], mix_v)
    pltpu.sync_copy(par_hbm, p_v)

    copies = [pltpu.async_copy(feat_hbm.at[pl.ds(base, BPW)], f_v, sem)]
    for j in range(ROWS_PER_W):
        copies.append(pltpu.async_copy(
            utab_hbm.at[uix_v.at[pl.ds(j * IDX_W, IDX_W)]],
            u_v.at[pl.ds(j * IDX_W, IDX_W)], sem))
        copies.append(pltpu.async_copy(
            mtab_hbm.at[mix_v.at[pl.ds(j * IDX_W, IDX_W)]],
            m_v.at[pl.ds(j * IDX_W, IDX_W)], sem))
    for c in copies:
        c.wait()

    w0 = p_v[pl.ds(0, 16)]
    w1 = p_v[pl.ds(16, 16)]
    wf0 = p_v[pl.ds(32, 16)]
    wf1 = p_v[pl.ds(48, 16)]
    b_vec = p_v[pl.ds(64, 16)]
    lane = lax.iota(jnp.int32, 16)

    def lanesum(t):
        for k in (1, 2, 4, 8):
            t = t + t.at[lane ^ k].get(mode="promise_in_bounds",
                                       unique_indices=True)
        return t

    def group(g, carry):
        base_i = g * 16
        acc = jnp.zeros((16,), jnp.float32)
        for j in range(16):
            i = base_i + j
            u0 = u_v[i, pl.ds(0, 16)]
            u1 = u_v[i, pl.ds(16, 16)]
            m0 = m_v[i, pl.ds(0, 16)]
            m1 = m_v[i, pl.ds(16, 16)]
            f0 = f_v[i, pl.ds(0, 16)]
            f1 = f_v[i, pl.ds(16, 16)]
            t = u0 * m0 * w0 + u1 * m1 * w1 + f0 * wf0 + f1 * wf1
            acc = jnp.where(lane == j, lanesum(t), acc)
        o_v[pl.ds(base_i, 16)] = acc + b_vec
        return carry

    lax.fori_loop(0, BPW // 16, group, 0)

    pltpu.sync_copy(o_v, out_hbm.at[pl.ds(base, BPW)])


_sc_call = functools.partial(
    pl.kernel,
    mesh=plsc.VectorSubcoreMesh(core_axis_name="c", subcore_axis_name="s"),
    out_type=jax.ShapeDtypeStruct((BATCH,), jnp.float32),
    compiler_params=pltpu.CompilerParams(use_tc_tiling_on_sc=False),
    scratch_types=[
        pltpu.VMEM((BPW,), jnp.int32),
        pltpu.VMEM((BPW,), jnp.int32),
        pltpu.VMEM((BPW, DIM), jnp.float32),
        pltpu.VMEM((BPW, DIM), jnp.float32),
        pltpu.VMEM((BPW, DIM), jnp.float32),
        pltpu.VMEM((NPARAM,), jnp.float32),
        pltpu.VMEM((BPW,), jnp.float32),
        pltpu.SemaphoreType.DMA,
    ],
)(_sc_body)


def kernel(user_idx, movie_idx, features, user_table, movie_table, fc_w, fc_b):
    uix = user_idx.astype(jnp.int32)
    mix = movie_idx.astype(jnp.int32)
    featp = jnp.pad(features, ((0, 0), (0, DIM - NFEAT)))
    w = fc_w.reshape(-1)
    params = jnp.concatenate([
        w[:DIM],
        jnp.pad(w[DIM:], (0, DIM - NFEAT)),
        jnp.broadcast_to(fc_b.reshape(-1), (16,)),
    ])
    return _sc_call(uix, mix, featp, user_table, movie_table, params)
